# baseline (speedup 1.0000x reference)
import jax
import jax.numpy as jnp
from jax.experimental import pallas as pl
from jax.experimental.pallas import tpu as pltpu

N = 8192
D = 128
C = 1024
NC = N // C
INV = 1.0 / (N - 1)


def _acc_body(nodes_ref, w2_ref, b2_ref, w3_ref, b3_ref, w5_ref, b5_ref,
              u_ref, c_ref, w25_ref, c25_ref, s_ref, hs_ref):
    k = pl.program_id(0)

    @pl.when(k == 0)
    def _init():
        s_ref[:] = jnp.zeros((D, D), jnp.float32)
        hs_ref[:] = jnp.zeros((1, D), jnp.float32)

    nc = nodes_ref[:]
    hc = jax.lax.dot_general(
        nc, w3_ref[:], (((1,), (1,)), ((), ())),
        preferred_element_type=jnp.float32) + b3_ref[:]
    s_ref[:] += jax.lax.dot_general(
        hc, nc, (((0,), (0,)), ((), ())),
        preferred_element_type=jnp.float32)
    hs_ref[:] += jnp.sum(hc, axis=0, keepdims=True)

    @pl.when(k == NC - 1)
    def _finalize():
        w25 = jax.lax.dot_general(
            w5_ref[:], w2_ref[:], (((1,), (0,)), ((), ())),
            preferred_element_type=jnp.float32)
        c25 = jax.lax.dot_general(
            b2_ref[:], w5_ref[:], (((1,), (1,)), ((), ())),
            preferred_element_type=jnp.float32)
        w25_ref[:] = w25
        c25_ref[:] = c25
        t = jax.lax.dot_general(
            s_ref[:], w25, (((1,), (1,)), ((), ())),
            preferred_element_type=jnp.float32) + jax.lax.dot_general(
            hs_ref[:], c25, (((0,), (0,)), ((), ())),
            preferred_element_type=jnp.float32)
        u_ref[:] = jax.lax.dot_general(
            w3_ref[:], t, (((0,), (0,)), ((), ())),
            preferred_element_type=jnp.float32) * INV
        c_ref[:] = jax.lax.dot_general(
            b3_ref[:], t, (((1,), (0,)), ((), ())),
            preferred_element_type=jnp.float32) * INV + b5_ref[:]


def _out_body(nodes_ref, w3_ref, b3_ref, u_ref, c_ref, w25_ref, c25_ref,
              out_ref):
    nc = nodes_ref[:]
    hc = jax.lax.dot_general(
        nc, w3_ref[:], (((1,), (1,)), ((), ())),
        preferred_element_type=jnp.float32) + b3_ref[:]
    d = jnp.sum(hc * hc, axis=1, keepdims=True) * INV
    out_ref[:] = jax.lax.dot_general(
        nc, u_ref[:], (((1,), (0,)), ((), ())),
        preferred_element_type=jnp.float32)
    out_ref[:] -= jax.lax.dot_general(
        d * nc, w25_ref[:], (((1,), (1,)), ((), ())),
        preferred_element_type=jnp.float32)
    out_ref[:] += nc + c_ref[:] - d * c25_ref[:]


@jax.jit
def kernel(nodes_in, inputs, W2, b2, W3, b3, W5, b5):
    del inputs
    b2r = b2.reshape(1, D)
    b3r = b3.reshape(1, D)
    b5r = b5.reshape(1, D)

    row_block = pl.BlockSpec((C, D), lambda i: (i, 0))
    full_dd = pl.BlockSpec((D, D), lambda i: (0, 0))
    full_1d = pl.BlockSpec((1, D), lambda i: (0, 0))

    u, c, w25, c25 = pl.pallas_call(
        _acc_body,
        grid=(NC,),
        in_specs=[row_block, full_dd, full_1d, full_dd, full_1d,
                  full_dd, full_1d],
        out_specs=[full_dd, full_1d, full_dd, full_1d],
        out_shape=[
            jax.ShapeDtypeStruct((D, D), jnp.float32),
            jax.ShapeDtypeStruct((1, D), jnp.float32),
            jax.ShapeDtypeStruct((D, D), jnp.float32),
            jax.ShapeDtypeStruct((1, D), jnp.float32),
        ],
        scratch_shapes=[
            pltpu.VMEM((D, D), jnp.float32),
            pltpu.VMEM((1, D), jnp.float32),
        ],
    )(nodes_in, W2, b2r, W3, b3r, W5, b5r)

    return pl.pallas_call(
        _out_body,
        grid=(NC,),
        in_specs=[row_block, full_dd, full_1d, full_dd, full_1d,
                  full_dd, full_1d],
        out_specs=row_block,
        out_shape=jax.ShapeDtypeStruct((N, D), jnp.float32),
    )(nodes_in, W3, b3r, u, c, w25, c25)


# speedup vs baseline: 1.1043x; 1.1043x over previous
import jax
import jax.numpy as jnp
from jax.experimental import pallas as pl
from jax.experimental.pallas import tpu as pltpu

N = 8192
D = 128
C = 1024
NC = N // C
INV = 1.0 / (N - 1)


def _body(nodes_ref, w2_ref, b2_ref, w3_ref, b3_ref, w5_ref, b5_ref,
          out_ref, s_ref, hs_ref, u_ref, c_ref, w25_ref, c25_ref, d_ref):
    p = pl.program_id(0)
    i = pl.program_id(1)

    @pl.when((p == 0) & (i == 0))
    def _init():
        s_ref[:] = jnp.zeros((D, D), jnp.float32)
        hs_ref[:] = jnp.zeros((1, D), jnp.float32)

    @pl.when(p == 0)
    def _accumulate():
        nc = nodes_ref[:]
        hc = jax.lax.dot_general(
            nc, w3_ref[:], (((1,), (1,)), ((), ())),
            preferred_element_type=jnp.float32) + b3_ref[:]
        s_ref[:] += jax.lax.dot_general(
            hc, nc, (((0,), (0,)), ((), ())),
            preferred_element_type=jnp.float32)
        hs_ref[:] += jnp.sum(hc, axis=0, keepdims=True)
        d_ref[pl.ds(i * C, C), :] = jnp.sum(
            hc * hc, axis=1, keepdims=True) * INV

    @pl.when((p == 0) & (i == NC - 1))
    def _finalize():
        w25 = jax.lax.dot_general(
            w5_ref[:], w2_ref[:], (((1,), (0,)), ((), ())),
            preferred_element_type=jnp.float32)
        c25 = jax.lax.dot_general(
            b2_ref[:], w5_ref[:], (((1,), (1,)), ((), ())),
            preferred_element_type=jnp.float32)
        w25_ref[:] = w25
        c25_ref[:] = c25
        t = jax.lax.dot_general(
            s_ref[:], w25, (((1,), (1,)), ((), ())),
            preferred_element_type=jnp.float32) + jax.lax.dot_general(
            hs_ref[:], c25, (((0,), (0,)), ((), ())),
            preferred_element_type=jnp.float32)
        u_ref[:] = jax.lax.dot_general(
            w3_ref[:], t, (((0,), (0,)), ((), ())),
            preferred_element_type=jnp.float32) * INV
        c_ref[:] = jax.lax.dot_general(
            b3_ref[:], t, (((1,), (0,)), ((), ())),
            preferred_element_type=jnp.float32) * INV + b5_ref[:]

    @pl.when(p == 1)
    def _emit():
        nc = nodes_ref[:]
        d = d_ref[pl.ds(i * C, C), :]
        out_ref[:] = jax.lax.dot_general(
            nc, u_ref[:], (((1,), (0,)), ((), ())),
            preferred_element_type=jnp.float32)
        out_ref[:] -= jax.lax.dot_general(
            d * nc, w25_ref[:], (((1,), (1,)), ((), ())),
            preferred_element_type=jnp.float32)
        out_ref[:] += nc + c_ref[:] - d * c25_ref[:]


@jax.jit
def kernel(nodes_in, inputs, W2, b2, W3, b3, W5, b5):
    del inputs
    row_block = pl.BlockSpec((C, D), lambda p, i: (i, 0))
    out_block = pl.BlockSpec(
        (C, D), lambda p, i: (jnp.where(p == 1, i, 0), 0))
    full_dd = pl.BlockSpec((D, D), lambda p, i: (0, 0))
    full_1d = pl.BlockSpec((1, D), lambda p, i: (0, 0))

    return pl.pallas_call(
        _body,
        grid=(2, NC),
        in_specs=[row_block, full_dd, full_1d, full_dd, full_1d,
                  full_dd, full_1d],
        out_specs=out_block,
        out_shape=jax.ShapeDtypeStruct((N, D), jnp.float32),
        scratch_shapes=[
            pltpu.VMEM((D, D), jnp.float32),
            pltpu.VMEM((1, D), jnp.float32),
            pltpu.VMEM((D, D), jnp.float32),
            pltpu.VMEM((1, D), jnp.float32),
            pltpu.VMEM((D, D), jnp.float32),
            pltpu.VMEM((1, D), jnp.float32),
            pltpu.VMEM((N, 1), jnp.float32),
        ],
    )(nodes_in, W2, b2.reshape(1, D), W3, b3.reshape(1, D),
      W5, b5.reshape(1, D))


# speedup vs baseline: 1.5462x; 1.4002x over previous
import jax
import jax.numpy as jnp
from jax.experimental import pallas as pl
from jax.experimental.pallas import tpu as pltpu

N = 8192
D = 128
C = 2048
NC = N // C
INV = 1.0 / (N - 1)


def _body(nodes_ref, w2_ref, b2_ref, w3_ref, b3_ref, w5_ref, b5_ref,
          out_ref, s_ref, hs_ref, u_ref, c_ref, w25_ref, c25_ref, d_ref):
    p = pl.program_id(0)
    i = pl.program_id(1)

    @pl.when((p == 0) & (i == 0))
    def _init():
        s_ref[:] = jnp.zeros((D, D), jnp.float32)
        hs_ref[:] = jnp.zeros((1, D), jnp.float32)

    @pl.when(p == 0)
    def _accumulate():
        nc = nodes_ref[:]
        hc = jax.lax.dot_general(
            nc, w3_ref[:], (((1,), (1,)), ((), ())),
            preferred_element_type=jnp.float32) + b3_ref[:]
        s_ref[:] += jax.lax.dot_general(
            hc, nc, (((0,), (0,)), ((), ())),
            preferred_element_type=jnp.float32)
        hs_ref[:] += jnp.sum(hc, axis=0, keepdims=True)
        d_ref[pl.ds(i * C, C), :] = jnp.sum(
            hc * hc, axis=1, keepdims=True) * INV

    @pl.when((p == 0) & (i == NC - 1))
    def _finalize():
        w25 = jax.lax.dot_general(
            w5_ref[:], w2_ref[:], (((1,), (0,)), ((), ())),
            preferred_element_type=jnp.float32)
        c25 = jax.lax.dot_general(
            b2_ref[:], w5_ref[:], (((1,), (1,)), ((), ())),
            preferred_element_type=jnp.float32)
        w25_ref[:] = w25
        c25_ref[:] = c25
        t = jax.lax.dot_general(
            s_ref[:], w25, (((1,), (1,)), ((), ())),
            preferred_element_type=jnp.float32) + jax.lax.dot_general(
            hs_ref[:], c25, (((0,), (0,)), ((), ())),
            preferred_element_type=jnp.float32)
        u_ref[:] = jax.lax.dot_general(
            w3_ref[:], t, (((0,), (0,)), ((), ())),
            preferred_element_type=jnp.float32) * INV
        c_ref[:] = jax.lax.dot_general(
            b3_ref[:], t, (((1,), (0,)), ((), ())),
            preferred_element_type=jnp.float32) * INV + b5_ref[:]

    @pl.when(p == 1)
    def _emit():
        nc = nodes_ref[:]
        d = d_ref[pl.ds(i * C, C), :]
        out_ref[:] = jax.lax.dot_general(
            nc, u_ref[:], (((1,), (0,)), ((), ())),
            preferred_element_type=jnp.float32)
        out_ref[:] -= jax.lax.dot_general(
            d * nc, w25_ref[:], (((1,), (1,)), ((), ())),
            preferred_element_type=jnp.float32)
        out_ref[:] += nc + c_ref[:] - d * c25_ref[:]


@jax.jit
def kernel(nodes_in, inputs, W2, b2, W3, b3, W5, b5):
    del inputs
    row_block = pl.BlockSpec((C, D), lambda p, i: (i, 0))
    out_block = pl.BlockSpec(
        (C, D), lambda p, i: (jnp.where(p == 1, i, 0), 0))
    full_dd = pl.BlockSpec((D, D), lambda p, i: (0, 0))
    full_1d = pl.BlockSpec((1, D), lambda p, i: (0, 0))

    return pl.pallas_call(
        _body,
        grid=(2, NC),
        in_specs=[row_block, full_dd, full_1d, full_dd, full_1d,
                  full_dd, full_1d],
        out_specs=out_block,
        out_shape=jax.ShapeDtypeStruct((N, D), jnp.float32),
        scratch_shapes=[
            pltpu.VMEM((D, D), jnp.float32),
            pltpu.VMEM((1, D), jnp.float32),
            pltpu.VMEM((D, D), jnp.float32),
            pltpu.VMEM((1, D), jnp.float32),
            pltpu.VMEM((D, D), jnp.float32),
            pltpu.VMEM((1, D), jnp.float32),
            pltpu.VMEM((N, 1), jnp.float32),
        ],
    )(nodes_in, W2, b2.reshape(1, D), W3, b3.reshape(1, D),
      W5, b5.reshape(1, D))


# speedup vs baseline: 1.9069x; 1.2333x over previous
import jax
import jax.numpy as jnp
from jax.experimental import pallas as pl
from jax.experimental.pallas import tpu as pltpu

N = 8192
D = 128
C = 4096
NC = N // C
INV = 1.0 / (N - 1)


def _body(nodes_ref, w2_ref, b2_ref, w3_ref, b3_ref, w5_ref, b5_ref,
          out_ref, s_ref, hs_ref, u_ref, c_ref, w25_ref, c25_ref, d_ref):
    p = pl.program_id(0)
    i = pl.program_id(1)

    @pl.when((p == 0) & (i == 0))
    def _init():
        s_ref[:] = jnp.zeros((D, D), jnp.float32)
        hs_ref[:] = jnp.zeros((1, D), jnp.float32)

    @pl.when(p == 0)
    def _accumulate():
        nc = nodes_ref[:]
        hc = jax.lax.dot_general(
            nc, w3_ref[:], (((1,), (1,)), ((), ())),
            preferred_element_type=jnp.float32) + b3_ref[:]
        s_ref[:] += jax.lax.dot_general(
            hc, nc, (((0,), (0,)), ((), ())),
            preferred_element_type=jnp.float32)
        hs_ref[:] += jnp.sum(hc, axis=0, keepdims=True)
        d_ref[pl.ds(i * C, C), :] = jnp.sum(
            hc * hc, axis=1, keepdims=True) * INV

    @pl.when((p == 0) & (i == NC - 1))
    def _finalize():
        w25 = jax.lax.dot_general(
            w5_ref[:], w2_ref[:], (((1,), (0,)), ((), ())),
            preferred_element_type=jnp.float32)
        c25 = jax.lax.dot_general(
            b2_ref[:], w5_ref[:], (((1,), (1,)), ((), ())),
            preferred_element_type=jnp.float32)
        w25_ref[:] = w25
        c25_ref[:] = c25
        t = jax.lax.dot_general(
            s_ref[:], w25, (((1,), (1,)), ((), ())),
            preferred_element_type=jnp.float32) + jax.lax.dot_general(
            hs_ref[:], c25, (((0,), (0,)), ((), ())),
            preferred_element_type=jnp.float32)
        u_ref[:] = jax.lax.dot_general(
            w3_ref[:], t, (((0,), (0,)), ((), ())),
            preferred_element_type=jnp.float32) * INV
        c_ref[:] = jax.lax.dot_general(
            b3_ref[:], t, (((1,), (0,)), ((), ())),
            preferred_element_type=jnp.float32) * INV + b5_ref[:]

    @pl.when(p == 1)
    def _emit():
        nc = nodes_ref[:]
        d = d_ref[pl.ds(i * C, C), :]
        out_ref[:] = jax.lax.dot_general(
            nc, u_ref[:], (((1,), (0,)), ((), ())),
            preferred_element_type=jnp.float32)
        out_ref[:] -= jax.lax.dot_general(
            d * nc, w25_ref[:], (((1,), (1,)), ((), ())),
            preferred_element_type=jnp.float32)
        out_ref[:] += nc + c_ref[:] - d * c25_ref[:]


@jax.jit
def kernel(nodes_in, inputs, W2, b2, W3, b3, W5, b5):
    del inputs
    row_block = pl.BlockSpec((C, D), lambda p, i: (i, 0))
    out_block = pl.BlockSpec(
        (C, D), lambda p, i: (jnp.where(p == 1, i, 0), 0))
    full_dd = pl.BlockSpec((D, D), lambda p, i: (0, 0))
    full_1d = pl.BlockSpec((1, D), lambda p, i: (0, 0))

    return pl.pallas_call(
        _body,
        grid=(2, NC),
        in_specs=[row_block, full_dd, full_1d, full_dd, full_1d,
                  full_dd, full_1d],
        out_specs=out_block,
        out_shape=jax.ShapeDtypeStruct((N, D), jnp.float32),
        scratch_shapes=[
            pltpu.VMEM((D, D), jnp.float32),
            pltpu.VMEM((1, D), jnp.float32),
            pltpu.VMEM((D, D), jnp.float32),
            pltpu.VMEM((1, D), jnp.float32),
            pltpu.VMEM((D, D), jnp.float32),
            pltpu.VMEM((1, D), jnp.float32),
            pltpu.VMEM((N, 1), jnp.float32),
        ],
    )(nodes_in, W2, b2.reshape(1, D), W3, b3.reshape(1, D),
      W5, b5.reshape(1, D))


# speedup vs baseline: 1.9922x; 1.0447x over previous
import jax
import jax.numpy as jnp
from jax.experimental import pallas as pl
from jax.experimental.pallas import tpu as pltpu

N = 8192
D = 128
C = 8192
NC = N // C
INV = 1.0 / (N - 1)


def _body(nodes_ref, w2_ref, b2_ref, w3_ref, b3_ref, w5_ref, b5_ref,
          out_ref, s_ref, hs_ref, u_ref, c_ref, w25_ref, c25_ref, d_ref):
    p = pl.program_id(0)
    i = pl.program_id(1)

    @pl.when((p == 0) & (i == 0))
    def _init():
        s_ref[:] = jnp.zeros((D, D), jnp.float32)
        hs_ref[:] = jnp.zeros((1, D), jnp.float32)

    @pl.when(p == 0)
    def _accumulate():
        nc = nodes_ref[:]
        hc = jax.lax.dot_general(
            nc, w3_ref[:], (((1,), (1,)), ((), ())),
            preferred_element_type=jnp.float32) + b3_ref[:]
        s_ref[:] += jax.lax.dot_general(
            hc, nc, (((0,), (0,)), ((), ())),
            preferred_element_type=jnp.float32)
        hs_ref[:] += jnp.sum(hc, axis=0, keepdims=True)
        d_ref[pl.ds(i * C, C), :] = jnp.sum(
            hc * hc, axis=1, keepdims=True) * INV

    @pl.when((p == 0) & (i == NC - 1))
    def _finalize():
        w25 = jax.lax.dot_general(
            w5_ref[:], w2_ref[:], (((1,), (0,)), ((), ())),
            preferred_element_type=jnp.float32)
        c25 = jax.lax.dot_general(
            b2_ref[:], w5_ref[:], (((1,), (1,)), ((), ())),
            preferred_element_type=jnp.float32)
        w25_ref[:] = w25
        c25_ref[:] = c25
        t = jax.lax.dot_general(
            s_ref[:], w25, (((1,), (1,)), ((), ())),
            preferred_element_type=jnp.float32) + jax.lax.dot_general(
            hs_ref[:], c25, (((0,), (0,)), ((), ())),
            preferred_element_type=jnp.float32)
        u_ref[:] = jax.lax.dot_general(
            w3_ref[:], t, (((0,), (0,)), ((), ())),
            preferred_element_type=jnp.float32) * INV
        c_ref[:] = jax.lax.dot_general(
            b3_ref[:], t, (((1,), (0,)), ((), ())),
            preferred_element_type=jnp.float32) * INV + b5_ref[:]

    @pl.when(p == 1)
    def _emit():
        nc = nodes_ref[:]
        d = d_ref[pl.ds(i * C, C), :]
        out_ref[:] = jax.lax.dot_general(
            nc, u_ref[:], (((1,), (0,)), ((), ())),
            preferred_element_type=jnp.float32)
        out_ref[:] -= jax.lax.dot_general(
            d * nc, w25_ref[:], (((1,), (1,)), ((), ())),
            preferred_element_type=jnp.float32)
        out_ref[:] += nc + c_ref[:] - d * c25_ref[:]


@jax.jit
def kernel(nodes_in, inputs, W2, b2, W3, b3, W5, b5):
    del inputs
    row_block = pl.BlockSpec((C, D), lambda p, i: (i, 0))
    out_block = pl.BlockSpec(
        (C, D), lambda p, i: (jnp.where(p == 1, i, 0), 0))
    full_dd = pl.BlockSpec((D, D), lambda p, i: (0, 0))
    full_1d = pl.BlockSpec((1, D), lambda p, i: (0, 0))

    return pl.pallas_call(
        _body,
        grid=(2, NC),
        in_specs=[row_block, full_dd, full_1d, full_dd, full_1d,
                  full_dd, full_1d],
        out_specs=out_block,
        out_shape=jax.ShapeDtypeStruct((N, D), jnp.float32),
        scratch_shapes=[
            pltpu.VMEM((D, D), jnp.float32),
            pltpu.VMEM((1, D), jnp.float32),
            pltpu.VMEM((D, D), jnp.float32),
            pltpu.VMEM((1, D), jnp.float32),
            pltpu.VMEM((D, D), jnp.float32),
            pltpu.VMEM((1, D), jnp.float32),
            pltpu.VMEM((N, 1), jnp.float32),
        ],
    )(nodes_in, W2, b2.reshape(1, D), W3, b3.reshape(1, D),
      W5, b5.reshape(1, D))


# speedup vs baseline: 1.9987x; 1.0032x over previous
import jax
import jax.numpy as jnp
from jax.experimental import pallas as pl
from jax.experimental.pallas import tpu as pltpu

N = 8192
D = 128
C = 8192
NC = N // C
INV = 1.0 / (N - 1)


def _body(nodes_ref, w2_ref, b2_ref, w3_ref, b3_ref, w5_ref, b5_ref,
          out_ref, s_ref, hs_ref, u_ref, c_ref, w25_ref, c25_ref, d_ref):
    p = pl.program_id(0)
    i = pl.program_id(1)

    @pl.when((p == 0) & (i == 0))
    def _init():
        s_ref[:] = jnp.zeros((D, D), jnp.float32)
        hs_ref[:] = jnp.zeros((1, D), jnp.float32)

    @pl.when(p == 0)
    def _accumulate():
        nc = nodes_ref[:]
        ncb = nc.astype(jnp.bfloat16)
        hc = jax.lax.dot_general(
            ncb, w3_ref[:].astype(jnp.bfloat16), (((1,), (1,)), ((), ())),
            preferred_element_type=jnp.float32) + b3_ref[:]
        s_ref[:] += jax.lax.dot_general(
            hc.astype(jnp.bfloat16), ncb, (((0,), (0,)), ((), ())),
            preferred_element_type=jnp.float32)
        hs_ref[:] += jnp.sum(hc, axis=0, keepdims=True)
        d_ref[pl.ds(i * C, C), :] = jnp.sum(
            hc * hc, axis=1, keepdims=True) * INV

    @pl.when((p == 0) & (i == NC - 1))
    def _finalize():
        w25 = jax.lax.dot_general(
            w5_ref[:], w2_ref[:], (((1,), (0,)), ((), ())),
            preferred_element_type=jnp.float32)
        c25 = jax.lax.dot_general(
            b2_ref[:], w5_ref[:], (((1,), (1,)), ((), ())),
            preferred_element_type=jnp.float32)
        w25_ref[:] = w25
        c25_ref[:] = c25
        t = jax.lax.dot_general(
            s_ref[:], w25, (((1,), (1,)), ((), ())),
            preferred_element_type=jnp.float32) + jax.lax.dot_general(
            hs_ref[:], c25, (((0,), (0,)), ((), ())),
            preferred_element_type=jnp.float32)
        u_ref[:] = jax.lax.dot_general(
            w3_ref[:], t, (((0,), (0,)), ((), ())),
            preferred_element_type=jnp.float32) * INV
        c_ref[:] = jax.lax.dot_general(
            b3_ref[:], t, (((1,), (0,)), ((), ())),
            preferred_element_type=jnp.float32) * INV + b5_ref[:]

    @pl.when(p == 1)
    def _emit():
        nc = nodes_ref[:]
        d = d_ref[pl.ds(i * C, C), :]
        out_ref[:] = jax.lax.dot_general(
            nc.astype(jnp.bfloat16), u_ref[:].astype(jnp.bfloat16),
            (((1,), (0,)), ((), ())),
            preferred_element_type=jnp.float32)
        out_ref[:] -= jax.lax.dot_general(
            (d * nc).astype(jnp.bfloat16), w25_ref[:].astype(jnp.bfloat16),
            (((1,), (1,)), ((), ())),
            preferred_element_type=jnp.float32)
        out_ref[:] += nc + c_ref[:] - d * c25_ref[:]


@jax.jit
def kernel(nodes_in, inputs, W2, b2, W3, b3, W5, b5):
    del inputs
    row_block = pl.BlockSpec((C, D), lambda p, i: (i, 0))
    out_block = pl.BlockSpec(
        (C, D), lambda p, i: (jnp.where(p == 1, i, 0), 0))
    full_dd = pl.BlockSpec((D, D), lambda p, i: (0, 0))
    full_1d = pl.BlockSpec((1, D), lambda p, i: (0, 0))

    return pl.pallas_call(
        _body,
        grid=(2, NC),
        in_specs=[row_block, full_dd, full_1d, full_dd, full_1d,
                  full_dd, full_1d],
        out_specs=out_block,
        out_shape=jax.ShapeDtypeStruct((N, D), jnp.float32),
        scratch_shapes=[
            pltpu.VMEM((D, D), jnp.float32),
            pltpu.VMEM((1, D), jnp.float32),
            pltpu.VMEM((D, D), jnp.float32),
            pltpu.VMEM((1, D), jnp.float32),
            pltpu.VMEM((D, D), jnp.float32),
            pltpu.VMEM((1, D), jnp.float32),
            pltpu.VMEM((N, 1), jnp.float32),
        ],
    )(nodes_in, W2, b2.reshape(1, D), W3, b3.reshape(1, D),
      W5, b5.reshape(1, D))


# speedup vs baseline: 2.1337x; 1.0676x over previous
import jax
import jax.numpy as jnp
from jax.experimental import pallas as pl
from jax.experimental.pallas import tpu as pltpu

N = 8192
D = 128
INV = 1.0 / (N - 1)


def _body(nodes_ref, w2_ref, b2_ref, w3_ref, b3_ref, w5_ref, b5_ref,
          out_ref):
    nc = nodes_ref[:]
    ncb = nc.astype(jnp.bfloat16)
    hc = jax.lax.dot_general(
        ncb, w3_ref[:].astype(jnp.bfloat16), (((1,), (1,)), ((), ())),
        preferred_element_type=jnp.float32) + b3_ref[:]
    hcb = hc.astype(jnp.bfloat16)
    s = jax.lax.dot_general(
        hcb, ncb, (((0,), (0,)), ((), ())),
        preferred_element_type=jnp.float32)
    hs = jnp.sum(hc, axis=0, keepdims=True)
    hsq = (hc * hc).astype(jnp.bfloat16)
    d = jax.lax.dot_general(
        hsq, jnp.ones((D, D), jnp.bfloat16), (((1,), (0,)), ((), ())),
        preferred_element_type=jnp.float32) * INV

    w25 = jax.lax.dot_general(
        w5_ref[:], w2_ref[:], (((1,), (0,)), ((), ())),
        preferred_element_type=jnp.float32)
    c25 = jax.lax.dot_general(
        b2_ref[:], w5_ref[:], (((1,), (1,)), ((), ())),
        preferred_element_type=jnp.float32)
    t = jax.lax.dot_general(
        s, w25, (((1,), (1,)), ((), ())),
        preferred_element_type=jnp.float32) + jax.lax.dot_general(
        hs, c25, (((0,), (0,)), ((), ())),
        preferred_element_type=jnp.float32)
    u = jax.lax.dot_general(
        w3_ref[:], t, (((0,), (0,)), ((), ())),
        preferred_element_type=jnp.float32) * INV
    c = jax.lax.dot_general(
        b3_ref[:], t, (((1,), (0,)), ((), ())),
        preferred_element_type=jnp.float32) * INV + b5_ref[:]

    out_ref[:] = jax.lax.dot_general(
        ncb, u.astype(jnp.bfloat16), (((1,), (0,)), ((), ())),
        preferred_element_type=jnp.float32)
    out_ref[:] -= jax.lax.dot_general(
        (d * nc).astype(jnp.bfloat16), w25.astype(jnp.bfloat16),
        (((1,), (1,)), ((), ())),
        preferred_element_type=jnp.float32)
    out_ref[:] += nc + c - d * c25


@jax.jit
def kernel(nodes_in, inputs, W2, b2, W3, b3, W5, b5):
    del inputs
    full_nd = pl.BlockSpec((N, D), lambda: (0, 0))
    full_dd = pl.BlockSpec((D, D), lambda: (0, 0))
    full_1d = pl.BlockSpec((1, D), lambda: (0, 0))

    return pl.pallas_call(
        _body,
        grid=(),
        in_specs=[full_nd, full_dd, full_1d, full_dd, full_1d,
                  full_dd, full_1d],
        out_specs=full_nd,
        out_shape=jax.ShapeDtypeStruct((N, D), jnp.float32),
    )(nodes_in, W2, b2.reshape(1, D), W3, b3.reshape(1, D),
      W5, b5.reshape(1, D))


# speedup vs baseline: 2.1544x; 1.0097x over previous
import jax
import jax.numpy as jnp
from jax.experimental import pallas as pl
from jax.experimental.pallas import tpu as pltpu

N = 8192
D = 128
INV = 1.0 / (N - 1)


def _body(nodes_ref, w2_ref, b2_ref, w3_ref, b3_ref, w5_ref, b5_ref,
          out_ref):
    nc = nodes_ref[:]
    ncb = nc.astype(jnp.bfloat16)
    hc = jax.lax.dot_general(
        ncb, w3_ref[:].astype(jnp.bfloat16), (((1,), (1,)), ((), ())),
        preferred_element_type=jnp.float32) + b3_ref[:]
    hcb = hc.astype(jnp.bfloat16)
    s = jax.lax.dot_general(
        hcb, ncb, (((0,), (0,)), ((), ())),
        preferred_element_type=jnp.float32)
    hs = jnp.sum(hc, axis=0, keepdims=True)
    hsq = hcb * hcb
    d = jax.lax.dot_general(
        hsq, jnp.ones((D, D), jnp.bfloat16), (((1,), (0,)), ((), ())),
        preferred_element_type=jnp.float32)

    w25 = jax.lax.dot_general(
        w5_ref[:], w2_ref[:], (((1,), (0,)), ((), ())),
        preferred_element_type=jnp.float32)
    c25 = jax.lax.dot_general(
        b2_ref[:], w5_ref[:], (((1,), (1,)), ((), ())),
        preferred_element_type=jnp.float32)
    t = jax.lax.dot_general(
        s, w25, (((1,), (1,)), ((), ())),
        preferred_element_type=jnp.float32) + jax.lax.dot_general(
        hs, c25, (((0,), (0,)), ((), ())),
        preferred_element_type=jnp.float32)
    u = jax.lax.dot_general(
        w3_ref[:], t, (((0,), (0,)), ((), ())),
        preferred_element_type=jnp.float32) * INV
    c = jax.lax.dot_general(
        b3_ref[:], t, (((1,), (0,)), ((), ())),
        preferred_element_type=jnp.float32) * INV + b5_ref[:]

    lhs = jnp.concatenate([ncb, (d * nc).astype(jnp.bfloat16)], axis=1)
    rhs = jnp.concatenate(
        [u.astype(jnp.bfloat16), (w25 * -INV).T.astype(jnp.bfloat16)],
        axis=0)
    out_ref[:] = jax.lax.dot_general(
        lhs, rhs, (((1,), (0,)), ((), ())),
        preferred_element_type=jnp.float32)
    out_ref[:] += nc + c - d * (c25 * INV)


@jax.jit
def kernel(nodes_in, inputs, W2, b2, W3, b3, W5, b5):
    del inputs
    full_nd = pl.BlockSpec((N, D), lambda: (0, 0))
    full_dd = pl.BlockSpec((D, D), lambda: (0, 0))
    full_1d = pl.BlockSpec((1, D), lambda: (0, 0))

    return pl.pallas_call(
        _body,
        grid=(),
        in_specs=[full_nd, full_dd, full_1d, full_dd, full_1d,
                  full_dd, full_1d],
        out_specs=full_nd,
        out_shape=jax.ShapeDtypeStruct((N, D), jnp.float32),
    )(nodes_in, W2, b2.reshape(1, D), W3, b3.reshape(1, D),
      W5, b5.reshape(1, D))
